# baseline (speedup 1.0000x reference)
import functools

import jax
import jax.numpy as jnp
from jax import lax
from jax.experimental import pallas as pl
from jax.experimental.pallas import tpu as pltpu
from jax.experimental.pallas import tpu_sc as plsc

_NC = 2
_NS = 16
_NW = _NC * _NS
_LANES = 16

_BR = 128


def _router_body(x_ref, wg_ref, pos1_ref, pos2_ref, offs_ref, cnts_ref,
                 x1_ref, x2_ref, oh_ref, pref_ref):
    n, d = x_ref.shape
    e = wg_ref.shape[1]
    rb = 128
    nb = n // rb

    x = x_ref[...]
    logits = jnp.dot(x, wg_ref[...], preferred_element_type=jnp.float32)
    m = jnp.max(logits, axis=1, keepdims=True)
    ex = jnp.exp(logits - m)
    p = ex / jnp.sum(ex, axis=1, keepdims=True)

    ecols = lax.broadcasted_iota(jnp.int32, (n, e), 1)
    m1 = jnp.max(p, axis=1, keepdims=True)
    a1 = jnp.min(jnp.where(p == m1, ecols, e), axis=1, keepdims=True)
    oh1 = (ecols == a1).astype(jnp.float32)
    p2 = jnp.where(ecols == a1, -1.0, p)
    m2 = jnp.max(p2, axis=1, keepdims=True)
    a2 = jnp.min(jnp.where(p2 == m2, ecols, e), axis=1, keepdims=True)
    oh2 = (ecols == a2).astype(jnp.float32)

    oh_ref[...] = jnp.concatenate([oh1, oh2], axis=1)

    rr = lax.broadcasted_iota(jnp.int32, (rb, rb), 0)
    cc = lax.broadcasted_iota(jnp.int32, (rb, rb), 1)
    tstrict = (cc < rr).astype(jnp.float32)
    run = jnp.zeros((1, 2 * e), dtype=jnp.float32)
    for b in range(nb):
        ob = oh_ref[b * rb:(b + 1) * rb, :]
        pref_ref[b * rb:(b + 1) * rb, :] = run + jnp.dot(
            tstrict, ob, preferred_element_type=jnp.float32)
        run = run + jnp.sum(ob, axis=0, keepdims=True)

    cnt1 = run[:, :e]
    counts = cnt1 + run[:, e:]
    counts_pad = jnp.floor((counts + 7.0) / 8.0) * 8.0
    er = lax.broadcasted_iota(jnp.int32, (e, e), 0)
    ec = lax.broadcasted_iota(jnp.int32, (e, e), 1)
    ustrict = (er < ec).astype(jnp.float32)
    offs = jnp.dot(counts_pad, ustrict, preferred_element_type=jnp.float32)

    pref = pref_ref[...]
    rank1 = jnp.sum(pref[:, :e] * oh1, axis=1, keepdims=True)
    rank2 = jnp.sum(pref[:, e:] * oh2, axis=1, keepdims=True)
    base1 = jnp.sum(oh1 * offs, axis=1, keepdims=True)
    base2 = jnp.sum(oh2 * (offs + cnt1), axis=1, keepdims=True)
    pos1_ref[...] = (base1 + rank1).astype(jnp.int32)
    pos2_ref[...] = (base2 + rank2).astype(jnp.int32)
    offs_ref[...] = offs.astype(jnp.int32)
    cnts_ref[...] = counts.astype(jnp.int32)
    x1_ref[...] = x * m1
    x2_ref[...] = x * m2


def _run_router(xf, w_gate):
    n, d = xf.shape
    e = w_gate.shape[1]
    return pl.pallas_call(
        _router_body,
        out_shape=(
            jax.ShapeDtypeStruct((n, 1), jnp.int32),
            jax.ShapeDtypeStruct((n, 1), jnp.int32),
            jax.ShapeDtypeStruct((1, e), jnp.int32),
            jax.ShapeDtypeStruct((1, e), jnp.int32),
            jax.ShapeDtypeStruct((n, d), jnp.float32),
            jax.ShapeDtypeStruct((n, d), jnp.float32),
        ),
        scratch_shapes=[
            pltpu.VMEM((n, 2 * e), jnp.float32),
            pltpu.VMEM((n, 2 * e), jnp.float32),
        ],
    )(xf, w_gate)


def _dispatch_body(x1_hbm, x2_hbm, pos1_hbm, pos2_hbm, xs_hbm,
                   idx1_v, idx2_v, rows1_v, rows2_v, sem1, sem2):
    tpw = idx1_v.shape[0]
    wid = lax.axis_index("s") * _NC + lax.axis_index("c")
    base = wid * tpw
    pltpu.sync_copy(pos1_hbm.at[pl.ds(base, tpw)], idx1_v)
    pltpu.sync_copy(pos2_hbm.at[pl.ds(base, tpw)], idx2_v)
    pltpu.sync_copy(x1_hbm.at[pl.ds(base, tpw), :], rows1_v)
    pltpu.sync_copy(x2_hbm.at[pl.ds(base, tpw), :], rows2_v)
    cp1 = pltpu.async_copy(rows1_v, xs_hbm.at[idx1_v], sem1)
    cp2 = pltpu.async_copy(rows2_v, xs_hbm.at[idx2_v], sem2)
    cp1.wait()
    cp2.wait()


def _run_dispatch(x1, x2, pos1, pos2, e):
    n, d = x1.shape
    tpw = n // _NW
    tot_pad = 2 * n + 8 * e + _BR
    mesh = plsc.VectorSubcoreMesh(core_axis_name="c", subcore_axis_name="s",
                                  num_cores=_NC, num_subcores=_NS)
    return pl.kernel(
        _dispatch_body,
        out_type=jax.ShapeDtypeStruct((tot_pad, d), jnp.float32),
        mesh=mesh,
        scratch_types=[
            pltpu.VMEM((tpw,), jnp.int32),
            pltpu.VMEM((tpw,), jnp.int32),
            pltpu.VMEM((tpw, d), jnp.float32),
            pltpu.VMEM((tpw, d), jnp.float32),
            pltpu.SemaphoreType.DMA,
            pltpu.SemaphoreType.DMA,
        ],
    )(x1, x2, pos1, pos2)


_NBUF = 4
_NSPLIT = 2


def _gmm_body(offs_ref, cnts_ref, x_hbm, w1_hbm, w2_hbm, out_hbm,
              w1b, w2b, xin, oout, semw, semx, semo):
    e_total, d, h = w1_hbm.shape
    hs = d // _NSPLIT

    def w_copies(ei, j):
        cps = []
        for s in range(_NSPLIT):
            sl = pl.ds(s * hs, hs)
            cps.append(pltpu.make_async_copy(
                w1_hbm.at[ei, sl, :], w1b.at[j, sl, :], semw.at[j, 0, s]))
            cps.append(pltpu.make_async_copy(
                w2_hbm.at[ei, sl, :], w2b.at[j, sl, :], semw.at[j, 1, s]))
        return cps

    for j in range(_NBUF):
        for cp in w_copies(j, j):
            cp.start()

    def outer(i, carry):
        for j in range(_NBUF):
            ei = i * _NBUF + j
            for cp in w_copies(ei, j):
                cp.wait()
            off0 = offs_ref[0, ei]
            cnt = cnts_ref[0, ei]
            nb = (cnt + _BR - 1) // _BR

            def body(b, c2, j=j):
                start = pl.multiple_of(off0 + b * _BR, 8)
                cpi = pltpu.make_async_copy(
                    x_hbm.at[pl.ds(start, _BR), :], xin, semx)
                cpi.start()
                cpi.wait()
                hh = jnp.maximum(
                    jnp.dot(xin[...], w1b[j],
                            preferred_element_type=jnp.float32), 0.0)
                oout[...] = jnp.dot(hh, w2b[j],
                                    preferred_element_type=jnp.float32)
                cpo = pltpu.make_async_copy(
                    oout, out_hbm.at[pl.ds(start, _BR), :], semo)
                cpo.start()
                cpo.wait()
                return c2

            lax.fori_loop(0, nb, body, 0)

            @pl.when(ei + _NBUF < e_total)
            def _():
                for cp in w_copies(ei + _NBUF, j):
                    cp.start()
        return carry

    lax.fori_loop(0, e_total // _NBUF, outer, 0)


def _run_gmm(offs, cnts, x_sorted, w1, w2):
    e, d, h = w1.shape
    tot_pad = x_sorted.shape[0]
    return pl.pallas_call(
        _gmm_body,
        in_specs=[
            pl.BlockSpec(memory_space=pltpu.SMEM),
            pl.BlockSpec(memory_space=pltpu.SMEM),
            pl.BlockSpec(memory_space=pltpu.HBM),
            pl.BlockSpec(memory_space=pltpu.HBM),
            pl.BlockSpec(memory_space=pltpu.HBM),
        ],
        out_specs=pl.BlockSpec(memory_space=pltpu.HBM),
        scratch_shapes=[
            pltpu.VMEM((_NBUF, d, h), jnp.float32),
            pltpu.VMEM((_NBUF, h, d), jnp.float32),
            pltpu.VMEM((_BR, d), jnp.float32),
            pltpu.VMEM((_BR, d), jnp.float32),
            pltpu.SemaphoreType.DMA((_NBUF, 2, _NSPLIT)),
            pltpu.SemaphoreType.DMA,
            pltpu.SemaphoreType.DMA,
        ],
        out_shape=jax.ShapeDtypeStruct((tot_pad, d), jnp.float32),
    )(offs, cnts, x_sorted, w1, w2)


def _combine_body(outs_hbm, pos1_hbm, pos2_hbm, y_hbm,
                  idx1_v, idx2_v, rows1_v, rows2_v, sem1, sem2):
    tpw, d = rows1_v.shape
    nvec = d // _LANES
    wid = lax.axis_index("s") * _NC + lax.axis_index("c")
    tbase = wid * tpw
    pltpu.sync_copy(pos1_hbm.at[pl.ds(tbase, tpw)], idx1_v)
    pltpu.sync_copy(pos2_hbm.at[pl.ds(tbase, tpw)], idx2_v)
    cp1 = pltpu.async_copy(outs_hbm.at[idx1_v], rows1_v, sem1)
    cp2 = pltpu.async_copy(outs_hbm.at[idx2_v], rows2_v, sem2)
    cp1.wait()
    cp2.wait()

    def tok(i, c1):
        for v in range(nvec):
            sl = pl.ds(v * _LANES, _LANES)
            rows1_v[i, sl] = rows1_v[i, sl] + rows2_v[i, sl]
        return c1

    lax.fori_loop(0, tpw, tok, 0)
    pltpu.sync_copy(rows1_v, y_hbm.at[pl.ds(tbase, tpw), :])


def _run_combine(out_sorted, pos1, pos2, n):
    d = out_sorted.shape[1]
    tpw = n // _NW
    mesh = plsc.VectorSubcoreMesh(core_axis_name="c", subcore_axis_name="s",
                                  num_cores=_NC, num_subcores=_NS)
    return pl.kernel(
        _combine_body,
        out_type=jax.ShapeDtypeStruct((n, d), jnp.float32),
        mesh=mesh,
        scratch_types=[
            pltpu.VMEM((tpw,), jnp.int32),
            pltpu.VMEM((tpw,), jnp.int32),
            pltpu.VMEM((tpw, d), jnp.float32),
            pltpu.VMEM((tpw, d), jnp.float32),
            pltpu.SemaphoreType.DMA,
            pltpu.SemaphoreType.DMA,
        ],
    )(out_sorted, pos1, pos2)


def kernel(x, w_gate, w1, w2):
    b, s, d = x.shape
    n = b * s
    xf = x.reshape(n, d)

    pos1, pos2, offs, cnts, x1, x2 = _run_router(xf, w_gate)
    pos1 = pos1.reshape(n)
    pos2 = pos2.reshape(n)

    x_sorted = _run_dispatch(x1, x2, pos1, pos2, w1.shape[0])
    out_sorted = _run_gmm(offs, cnts, x_sorted, w1, w2)
    y = _run_combine(out_sorted, pos1, pos2, n)

    loss = jnp.zeros((), dtype=jnp.float32)
    return y.reshape(b, s, d), loss


# speedup vs baseline: 1.0088x; 1.0088x over previous
import functools

import jax
import jax.numpy as jnp
from jax import lax
from jax.experimental import pallas as pl
from jax.experimental.pallas import tpu as pltpu
from jax.experimental.pallas import tpu_sc as plsc

_NC = 2
_NS = 16
_NW = _NC * _NS
_LANES = 16

_BR = 128


def _router_body(x_ref, wg_ref, pos1_ref, pos2_ref, offs_ref, cnts_ref,
                 g1_ref, g2_ref, oh_ref, pref_ref):
    n, d = x_ref.shape
    e = wg_ref.shape[1]
    rb = 128
    nb = n // rb

    x = x_ref[...]
    logits = jnp.dot(x, wg_ref[...], preferred_element_type=jnp.float32)
    m = jnp.max(logits, axis=1, keepdims=True)
    ex = jnp.exp(logits - m)
    p = ex / jnp.sum(ex, axis=1, keepdims=True)

    ecols = lax.broadcasted_iota(jnp.int32, (n, e), 1)
    m1 = jnp.max(p, axis=1, keepdims=True)
    a1 = jnp.min(jnp.where(p == m1, ecols, e), axis=1, keepdims=True)
    oh1 = (ecols == a1).astype(jnp.float32)
    p2 = jnp.where(ecols == a1, -1.0, p)
    m2 = jnp.max(p2, axis=1, keepdims=True)
    a2 = jnp.min(jnp.where(p2 == m2, ecols, e), axis=1, keepdims=True)
    oh2 = (ecols == a2).astype(jnp.float32)

    oh_ref[...] = jnp.concatenate([oh1, oh2], axis=1)

    rr = lax.broadcasted_iota(jnp.int32, (rb, rb), 0)
    cc = lax.broadcasted_iota(jnp.int32, (rb, rb), 1)
    tstrict = (cc < rr).astype(jnp.float32)
    run = jnp.zeros((1, 2 * e), dtype=jnp.float32)
    for b in range(nb):
        ob = oh_ref[b * rb:(b + 1) * rb, :]
        pref_ref[b * rb:(b + 1) * rb, :] = run + jnp.dot(
            tstrict, ob, preferred_element_type=jnp.float32)
        run = run + jnp.sum(ob, axis=0, keepdims=True)

    cnt1 = run[:, :e]
    counts = cnt1 + run[:, e:]
    counts_pad = jnp.floor((counts + 7.0) / 8.0) * 8.0
    er = lax.broadcasted_iota(jnp.int32, (e, e), 0)
    ec = lax.broadcasted_iota(jnp.int32, (e, e), 1)
    ustrict = (er < ec).astype(jnp.float32)
    offs = jnp.dot(counts_pad, ustrict, preferred_element_type=jnp.float32)

    pref = pref_ref[...]
    rank1 = jnp.sum(pref[:, :e] * oh1, axis=1, keepdims=True)
    rank2 = jnp.sum(pref[:, e:] * oh2, axis=1, keepdims=True)
    base1 = jnp.sum(oh1 * offs, axis=1, keepdims=True)
    base2 = jnp.sum(oh2 * (offs + cnt1), axis=1, keepdims=True)
    pos1_ref[...] = (base1 + rank1).astype(jnp.int32)
    pos2_ref[...] = (base2 + rank2).astype(jnp.int32)
    offs_ref[...] = offs.astype(jnp.int32)
    cnts_ref[...] = counts.astype(jnp.int32)
    g1_ref[...] = jnp.broadcast_to(m1, (n, _LANES))
    g2_ref[...] = jnp.broadcast_to(m2, (n, _LANES))


def _run_router(xf, w_gate):
    n, d = xf.shape
    e = w_gate.shape[1]
    return pl.pallas_call(
        _router_body,
        out_shape=(
            jax.ShapeDtypeStruct((n, 1), jnp.int32),
            jax.ShapeDtypeStruct((n, 1), jnp.int32),
            jax.ShapeDtypeStruct((1, e), jnp.int32),
            jax.ShapeDtypeStruct((1, e), jnp.int32),
            jax.ShapeDtypeStruct((n, _LANES), jnp.float32),
            jax.ShapeDtypeStruct((n, _LANES), jnp.float32),
        ),
        scratch_shapes=[
            pltpu.VMEM((n, 2 * e), jnp.float32),
            pltpu.VMEM((n, 2 * e), jnp.float32),
        ],
    )(xf, w_gate)


def _dispatch_body(x_hbm, pos1_hbm, pos2_hbm, xs_hbm,
                   idx1_v, idx2_v, rows_v, sem1, sem2):
    tpw = idx1_v.shape[0]
    wid = lax.axis_index("s") * _NC + lax.axis_index("c")
    base = wid * tpw
    pltpu.sync_copy(pos1_hbm.at[pl.ds(base, tpw)], idx1_v)
    pltpu.sync_copy(pos2_hbm.at[pl.ds(base, tpw)], idx2_v)
    pltpu.sync_copy(x_hbm.at[pl.ds(base, tpw), :], rows_v)
    cp1 = pltpu.async_copy(rows_v, xs_hbm.at[idx1_v], sem1)
    cp2 = pltpu.async_copy(rows_v, xs_hbm.at[idx2_v], sem2)
    cp1.wait()
    cp2.wait()


def _run_dispatch(xf, pos1, pos2, e):
    n, d = xf.shape
    tpw = n // _NW
    tot_pad = 2 * n + 8 * e + _BR
    mesh = plsc.VectorSubcoreMesh(core_axis_name="c", subcore_axis_name="s",
                                  num_cores=_NC, num_subcores=_NS)
    return pl.kernel(
        _dispatch_body,
        out_type=jax.ShapeDtypeStruct((tot_pad, d), jnp.float32),
        mesh=mesh,
        scratch_types=[
            pltpu.VMEM((tpw,), jnp.int32),
            pltpu.VMEM((tpw,), jnp.int32),
            pltpu.VMEM((tpw, d), jnp.float32),
            pltpu.SemaphoreType.DMA,
            pltpu.SemaphoreType.DMA,
        ],
    )(xf, pos1, pos2)


_NBUF = 4
_NSPLIT = 2


def _gmm_body(offs_ref, cnts_ref, x_hbm, w1_hbm, w2_hbm, out_hbm,
              w1b, w2b, xin, oout, semw, semx, semo):
    e_total, d, h = w1_hbm.shape
    hs = d // _NSPLIT

    def w_copies(ei, j):
        cps = []
        for s in range(_NSPLIT):
            sl = pl.ds(s * hs, hs)
            cps.append(pltpu.make_async_copy(
                w1_hbm.at[ei, sl, :], w1b.at[j, sl, :], semw.at[j, 0, s]))
            cps.append(pltpu.make_async_copy(
                w2_hbm.at[ei, sl, :], w2b.at[j, sl, :], semw.at[j, 1, s]))
        return cps

    for j in range(_NBUF):
        for cp in w_copies(j, j):
            cp.start()

    def outer(i, carry):
        for j in range(_NBUF):
            ei = i * _NBUF + j
            for cp in w_copies(ei, j):
                cp.wait()
            off0 = offs_ref[0, ei]
            cnt = cnts_ref[0, ei]
            nb = (cnt + _BR - 1) // _BR

            def body(b, c2, j=j):
                start = pl.multiple_of(off0 + b * _BR, 8)
                cpi = pltpu.make_async_copy(
                    x_hbm.at[pl.ds(start, _BR), :], xin, semx)
                cpi.start()
                cpi.wait()
                hh = jnp.maximum(
                    jnp.dot(xin[...], w1b[j],
                            preferred_element_type=jnp.float32), 0.0)
                oout[...] = jnp.dot(hh, w2b[j],
                                    preferred_element_type=jnp.float32)
                cpo = pltpu.make_async_copy(
                    oout, out_hbm.at[pl.ds(start, _BR), :], semo)
                cpo.start()
                cpo.wait()
                return c2

            lax.fori_loop(0, nb, body, 0)

            @pl.when(ei + _NBUF < e_total)
            def _():
                for cp in w_copies(ei + _NBUF, j):
                    cp.start()
        return carry

    lax.fori_loop(0, e_total // _NBUF, outer, 0)


def _run_gmm(offs, cnts, x_sorted, w1, w2):
    e, d, h = w1.shape
    tot_pad = x_sorted.shape[0]
    return pl.pallas_call(
        _gmm_body,
        in_specs=[
            pl.BlockSpec(memory_space=pltpu.SMEM),
            pl.BlockSpec(memory_space=pltpu.SMEM),
            pl.BlockSpec(memory_space=pltpu.HBM),
            pl.BlockSpec(memory_space=pltpu.HBM),
            pl.BlockSpec(memory_space=pltpu.HBM),
        ],
        out_specs=pl.BlockSpec(memory_space=pltpu.HBM),
        scratch_shapes=[
            pltpu.VMEM((_NBUF, d, h), jnp.float32),
            pltpu.VMEM((_NBUF, h, d), jnp.float32),
            pltpu.VMEM((_BR, d), jnp.float32),
            pltpu.VMEM((_BR, d), jnp.float32),
            pltpu.SemaphoreType.DMA((_NBUF, 2, _NSPLIT)),
            pltpu.SemaphoreType.DMA,
            pltpu.SemaphoreType.DMA,
        ],
        out_shape=jax.ShapeDtypeStruct((tot_pad, d), jnp.float32),
    )(offs, cnts, x_sorted, w1, w2)


def _combine_body(outs_hbm, pos1_hbm, pos2_hbm, g1_hbm, g2_hbm, y_hbm,
                  idx1_v, idx2_v, g1_v, g2_v, rows1_v, rows2_v, sem1, sem2):
    tpw, d = rows1_v.shape
    nvec = d // _LANES
    wid = lax.axis_index("s") * _NC + lax.axis_index("c")
    tbase = wid * tpw
    pltpu.sync_copy(pos1_hbm.at[pl.ds(tbase, tpw)], idx1_v)
    pltpu.sync_copy(pos2_hbm.at[pl.ds(tbase, tpw)], idx2_v)
    pltpu.sync_copy(g1_hbm.at[pl.ds(tbase, tpw), :], g1_v)
    pltpu.sync_copy(g2_hbm.at[pl.ds(tbase, tpw), :], g2_v)
    cp1 = pltpu.async_copy(outs_hbm.at[idx1_v], rows1_v, sem1)
    cp2 = pltpu.async_copy(outs_hbm.at[idx2_v], rows2_v, sem2)
    cp1.wait()
    cp2.wait()

    def tok(i, c1):
        gv1 = g1_v[i, :]
        gv2 = g2_v[i, :]
        for v in range(nvec):
            sl = pl.ds(v * _LANES, _LANES)
            rows1_v[i, sl] = gv1 * rows1_v[i, sl] + gv2 * rows2_v[i, sl]
        return c1

    lax.fori_loop(0, tpw, tok, 0)
    pltpu.sync_copy(rows1_v, y_hbm.at[pl.ds(tbase, tpw), :])


def _run_combine(out_sorted, pos1, pos2, g1, g2, n):
    d = out_sorted.shape[1]
    tpw = n // _NW
    mesh = plsc.VectorSubcoreMesh(core_axis_name="c", subcore_axis_name="s",
                                  num_cores=_NC, num_subcores=_NS)
    return pl.kernel(
        _combine_body,
        out_type=jax.ShapeDtypeStruct((n, d), jnp.float32),
        mesh=mesh,
        scratch_types=[
            pltpu.VMEM((tpw,), jnp.int32),
            pltpu.VMEM((tpw,), jnp.int32),
            pltpu.VMEM((tpw, _LANES), jnp.float32),
            pltpu.VMEM((tpw, _LANES), jnp.float32),
            pltpu.VMEM((tpw, d), jnp.float32),
            pltpu.VMEM((tpw, d), jnp.float32),
            pltpu.SemaphoreType.DMA,
            pltpu.SemaphoreType.DMA,
        ],
    )(out_sorted, pos1, pos2, g1, g2)


def kernel(x, w_gate, w1, w2):
    b, s, d = x.shape
    n = b * s
    xf = x.reshape(n, d)

    pos1, pos2, offs, cnts, g1, g2 = _run_router(xf, w_gate)
    pos1 = pos1.reshape(n)
    pos2 = pos2.reshape(n)

    x_sorted = _run_dispatch(xf, pos1, pos2, w1.shape[0])
    out_sorted = _run_gmm(offs, cnts, x_sorted, w1, w2)
    y = _run_combine(out_sorted, pos1, pos2, g1, g2, n)

    loss = jnp.zeros((), dtype=jnp.float32)
    return y.reshape(b, s, d), loss
